# baseline (speedup 1.0000x reference)
import functools

import jax
import jax.numpy as jnp
from jax import lax
from jax.experimental import pallas as pl
from jax.experimental.pallas import tpu as pltpu
from jax.experimental.pallas import tpu_sc as plsc

N = 10000
E = 320000
L = 16
F = 128
H = 8
FO = 16
V = 1000
NLANES = 16
NC, NS = 2, 16
NPAD = 10240
NCH = NPAD // 64
EC = 64
CPT = 157
ERPAD = 5056
NSUP = 54
DH = F + 16


def _dense_body(xp, rp, lp, wp, asf, adf, fH, fE, rH, rE, lH, lE):
    w = wp[...]
    a_s = asf[...]
    a_d = adf[...]
    j = lax.broadcasted_iota(jnp.int32, (F, L), 0)
    l = lax.broadcasted_iota(jnp.int32, (F, L), 1)
    b16 = ((j % H) == (l % H)).astype(jnp.float32)
    we_s = jnp.dot((w * a_s), b16, preferred_element_type=jnp.float32)
    we_d = jnp.dot((w * a_d), b16, preferred_element_type=jnp.float32)
    we = jnp.concatenate(
        [we_s, we_d, jnp.zeros((F, F - 2 * L), jnp.float32)], axis=1)
    x = xp[...]
    r = rp[...]
    q = lp[...]
    fH[...] = jnp.dot(x, w, preferred_element_type=jnp.float32)
    fE[...] = jnp.dot(x, we, preferred_element_type=jnp.float32)
    rH[...] = jnp.dot(r, w, preferred_element_type=jnp.float32)
    rE[...] = jnp.dot(r, we, preferred_element_type=jnp.float32)
    lH[...] = jnp.dot(q, w, preferred_element_type=jnp.float32)
    lE[...] = jnp.dot(q, we, preferred_element_type=jnp.float32)


def _node_body(fH, fE, rH, rE, lH, lE, i0_2d, i1_2d, ht_out, ed_out,
               fHb, fEb, rHb, rEb, lHb, lEb, i0b, i1b, htb, edb, sems):
    cid = lax.axis_index("c")
    sid = lax.axis_index("s")
    wid = sid * NC + cid

    def do_chunk(c):
        base = c * 64
        pltpu.sync_copy(i0_2d.at[c], i0b)
        pltpu.sync_copy(i1_2d.at[c], i1b)
        cps = [
            pltpu.async_copy(fH.at[pl.ds(base, 64)], fHb, sems.at[0]),
            pltpu.async_copy(fE.at[pl.ds(base, 64)], fEb, sems.at[1]),
            pltpu.async_copy(rH.at[i0b], rHb, sems.at[2]),
            pltpu.async_copy(rE.at[i0b], rEb, sems.at[3]),
            pltpu.async_copy(lH.at[i1b], lHb, sems.at[4]),
            pltpu.async_copy(lE.at[i1b], lEb, sems.at[5]),
        ]
        for cp in cps:
            cp.wait()

        @plsc.parallel_loop(0, 64, unroll=2)
        def _(n):
            for g in range(H):
                s = pl.ds(g * 16, 16)
                htb[n, s] = fHb[n, s] + rHb[n, s] + lHb[n, s]
            se = pl.ds(0, 16)
            htb[n, pl.ds(F, 16)] = fEb[n, se] + rEb[n, se] + lEb[n, se]
            sd = pl.ds(16, 16)
            edb[n, :] = fEb[n, sd] + rEb[n, sd] + lEb[n, sd]
        pltpu.sync_copy(htb, ht_out.at[pl.ds(base, 64)])
        pltpu.sync_copy(edb, ed_out.at[pl.ds(base, 64)])

    for t in range(NCH // 32):
        do_chunk(wid + t * 32)


def _edge_body(ht, ed, src2d, dst2d, part_out, hb0, hb1, hb2, db0, db1, db2,
               sb0, sb1, dk0, dk1, acc, gh, ge, ss, gi):
    cid = lax.axis_index("c")
    sid = lax.axis_index("s")
    wid = sid * NC + cid
    ntile = jnp.minimum(E // EC - wid * CPT, CPT)
    rowbase = wid * CPT
    hbs = (hb0, hb1, hb2)
    dbs = (db0, db1, db2)
    sbs = (sb0, sb1)
    dks = (dk0, dk1)

    zrow = jnp.zeros((NLANES,), jnp.float32)

    def zero_row(n, _):
        for g in range(DH // 16):
            hb0[n, pl.ds(g * 16, 16)] = zrow
        return 0

    lax.fori_loop(0, EC, zero_row, 0)
    for k in range(NPAD // NS // EC):
        pltpu.sync_copy(hb0, acc.at[pl.ds(sid * 640 + k * EC, EC)])
    plsc.subcore_barrier()

    def load_block(s, b):
        pltpu.async_copy(src2d.at[pl.ds(rowbase + 3 * s, 3)], sbs[b],
                         gi.at[2 * b])
        pltpu.async_copy(dst2d.at[pl.ds(rowbase + 3 * s, 3)], dks[b],
                         gi.at[2 * b + 1])

    def wait_block(b):
        pltpu.make_async_copy(src2d.at[pl.ds(0, 3)], sbs[b],
                              gi.at[2 * b]).wait()
        pltpu.make_async_copy(dst2d.at[pl.ds(0, 3)], dks[b],
                              gi.at[2 * b + 1]).wait()

    def start_gathers(t, k, b, r):
        @pl.when(t < ntile)
        def _():
            pltpu.async_copy(ht.at[sbs[b].at[r]], hbs[k], gh.at[k])
            pltpu.async_copy(ed.at[dks[b].at[r]], dbs[k], ge.at[k])

    def step(t, k, b, r):
        hb, db = hbs[k], dbs[k]

        @pl.when(t < ntile)
        def _():
            pltpu.make_async_copy(ht.at[dk0.at[0]], hb, gh.at[k]).wait()
            pltpu.make_async_copy(ed.at[dk0.at[0]], db, ge.at[k]).wait()

            @plsc.parallel_loop(0, EC, unroll=4)
            def _(e):
                vs = hb[e, pl.ds(F, 16)]
                vd = db[e, :]
                x = vs + vd
                x = jnp.maximum(x, x * 0.2)
                ex = jnp.exp(x)
                hb[e, pl.ds(F, 16)] = ex
                for g in range(H):
                    s = pl.ds(g * 16, 16)
                    hb[e, s] = hb[e, s] * ex
            pltpu.async_copy(hb, acc.at[dks[b].at[r]], ss.at[k], add=True)

        kp2 = (k + 2) % 3

        @pl.when((t >= 1) & (t - 1 < ntile))
        def _():
            pltpu.make_async_copy(hbs[kp2], acc.at[dk0.at[0]],
                                  ss.at[kp2]).wait()

    load_block(0, 0)
    wait_block(0)
    start_gathers(0, 0, 0, 0)
    start_gathers(1, 1, 0, 1)

    def pairbody(u, _):
        t0 = u * 6
        step(t0 + 0, 0, 0, 0)
        load_block(2 * u + 1, 1)
        start_gathers(t0 + 2, 2, 0, 2)
        step(t0 + 1, 1, 0, 1)
        wait_block(1)
        start_gathers(t0 + 3, 0, 1, 0)
        step(t0 + 2, 2, 0, 2)
        start_gathers(t0 + 4, 1, 1, 1)
        step(t0 + 3, 0, 1, 0)
        load_block(2 * u + 2, 0)
        start_gathers(t0 + 5, 2, 1, 2)
        step(t0 + 4, 1, 1, 1)
        wait_block(0)
        start_gathers(t0 + 6, 0, 0, 0)
        step(t0 + 5, 2, 1, 2)
        start_gathers(t0 + 7, 1, 0, 1)
        return 0

    lax.fori_loop(0, NSUP // 2, pairbody, 0)

    plsc.subcore_barrier()
    pltpu.sync_copy(acc.at[pl.ds(sid * 640, 640)],
                    part_out.at[cid, pl.ds(sid * 640, 640)])


def _finish_body(part, out):
    p = part[0] + part[1]
    num = p[:, :F]
    d16 = p[:, F:DH]
    i16 = lax.broadcasted_iota(jnp.int32, (L, F), 0)
    j = lax.broadcasted_iota(jnp.int32, (L, F), 1)
    r16 = ((i16 == (j % H)) & (i16 < H)).astype(jnp.float32)
    dd = jnp.dot(d16, r16, preferred_element_type=jnp.float32)
    a = num / (dd + 1e-16)
    jj = lax.broadcasted_iota(jnp.int32, (F, F), 0)
    mm = lax.broadcasted_iota(jnp.int32, (F, F), 1)
    perm = (jj == ((mm % FO) * H + mm // FO)).astype(jnp.float32)
    out[...] = jnp.maximum(jnp.dot(a, perm, preferred_element_type=jnp.float32),
                           0.0)


def kernel(lanes_feat, lane_ids, edge_index, road_emb, lane_emb, W, a_src,
           a_dst):
    f32 = jnp.float32
    lanes_feat = lanes_feat.astype(f32)
    wp = W.astype(f32).reshape(F, H, FO).transpose(0, 2, 1).reshape(F, F)
    asf = a_src.astype(f32).transpose(1, 0).reshape(1, F)
    adf = a_dst.astype(f32).transpose(1, 0).reshape(1, F)

    xp = jnp.zeros((NPAD, F), f32)
    xp = xp.at[:N, :L - 2].set(lanes_feat)
    rp = jnp.zeros((V, F), f32).at[:, L - 2:].set(road_emb.astype(f32))
    lpe = jnp.zeros((V, F), f32).at[:, L - 2:].set(lane_emb.astype(f32))

    ids = lane_ids.astype(jnp.int32)
    i0 = jnp.zeros((NPAD,), jnp.int32).at[:N].set(ids[:, 0]).reshape(NCH, 64)
    i1 = jnp.zeros((NPAD,), jnp.int32).at[:N].set(ids[:, 1]).reshape(NCH, 64)
    epad = ERPAD * EC - E
    src2d = jnp.pad(edge_index[0].astype(jnp.int32), (0, epad)).reshape(
        ERPAD, EC)
    dst2d = jnp.pad(edge_index[1].astype(jnp.int32), (0, epad)).reshape(
        ERPAD, EC)

    fH, fE, rH, rE, lH, lE = pl.pallas_call(
        _dense_body,
        out_shape=[
            jax.ShapeDtypeStruct((NPAD, F), f32),
            jax.ShapeDtypeStruct((NPAD, F), f32),
            jax.ShapeDtypeStruct((V, F), f32),
            jax.ShapeDtypeStruct((V, F), f32),
            jax.ShapeDtypeStruct((V, F), f32),
            jax.ShapeDtypeStruct((V, F), f32),
        ],
    )(xp, rp, lpe, wp, asf, adf)

    mesh = plsc.VectorSubcoreMesh(core_axis_name="c", subcore_axis_name="s")

    sc_params = pltpu.CompilerParams(use_tc_tiling_on_sc=False)
    node_k = pl.kernel(
        _node_body,
        out_type=[
            jax.ShapeDtypeStruct((NPAD, DH), f32),
            jax.ShapeDtypeStruct((NPAD, L), f32),
        ],
        mesh=mesh,
        compiler_params=sc_params,
        scratch_types=[
            pltpu.VMEM((64, F), f32),
            pltpu.VMEM((64, F), f32),
            pltpu.VMEM((64, F), f32),
            pltpu.VMEM((64, F), f32),
            pltpu.VMEM((64, F), f32),
            pltpu.VMEM((64, F), f32),
            pltpu.VMEM((64,), jnp.int32),
            pltpu.VMEM((64,), jnp.int32),
            pltpu.VMEM((64, DH), f32),
            pltpu.VMEM((64, L), f32),
            pltpu.SemaphoreType.DMA((6,)),
        ],
    )
    ht, edt = node_k(fH, fE, rH, rE, lH, lE, i0, i1)

    edge_k = pl.kernel(
        _edge_body,
        out_type=[jax.ShapeDtypeStruct((NC, NPAD, DH), f32)],
        mesh=mesh,
        compiler_params=sc_params,
        scratch_types=[
            pltpu.VMEM((EC, DH), f32),
            pltpu.VMEM((EC, DH), f32),
            pltpu.VMEM((EC, DH), f32),
            pltpu.VMEM((EC, L), f32),
            pltpu.VMEM((EC, L), f32),
            pltpu.VMEM((EC, L), f32),
            pltpu.VMEM((3, EC), jnp.int32),
            pltpu.VMEM((3, EC), jnp.int32),
            pltpu.VMEM((3, EC), jnp.int32),
            pltpu.VMEM((3, EC), jnp.int32),
            pltpu.VMEM_SHARED((NPAD, DH), f32),
            pltpu.SemaphoreType.DMA((3,)),
            pltpu.SemaphoreType.DMA((3,)),
            pltpu.SemaphoreType.DMA((3,)),
            pltpu.SemaphoreType.DMA((4,)),
        ],
    )
    (part,) = edge_k(ht, edt, src2d, dst2d)

    out = pl.pallas_call(
        _finish_body,
        grid=(NPAD // 512,),
        in_specs=[pl.BlockSpec((NC, 512, DH), lambda i: (0, i, 0))],
        out_specs=pl.BlockSpec((512, F), lambda i: (i, 0)),
        out_shape=jax.ShapeDtypeStruct((NPAD, F), f32),
    )(part)
    return out[:N]


# speedup vs baseline: 1.0029x; 1.0029x over previous
import functools

import jax
import jax.numpy as jnp
from jax import lax
from jax.experimental import pallas as pl
from jax.experimental.pallas import tpu as pltpu
from jax.experimental.pallas import tpu_sc as plsc

N = 10000
E = 320000
L = 16
F = 128
H = 8
FO = 16
V = 1000
NLANES = 16
NC, NS = 2, 16
NPAD = 10240
NCH = NPAD // 64
EC = 64
CPT = 157
ERPAD = 5056
NSUP = 54
DH = F + 16


def _dense_body(xp, rp, lp, wp, asf, adf, fH, fE, rH, rE, lH, lE):
    w = wp[...]
    a_s = asf[...]
    a_d = adf[...]
    j = lax.broadcasted_iota(jnp.int32, (F, L), 0)
    l = lax.broadcasted_iota(jnp.int32, (F, L), 1)
    b16 = ((j % H) == (l % H)).astype(jnp.float32)
    we_s = jnp.dot((w * a_s), b16, preferred_element_type=jnp.float32)
    we_d = jnp.dot((w * a_d), b16, preferred_element_type=jnp.float32)
    we = jnp.concatenate(
        [we_s, we_d, jnp.zeros((F, F - 2 * L), jnp.float32)], axis=1)
    x = xp[...]
    r = rp[...]
    q = lp[...]
    fH[...] = jnp.dot(x, w, preferred_element_type=jnp.float32)
    fE[...] = jnp.dot(x, we, preferred_element_type=jnp.float32)
    rH[...] = jnp.dot(r, w, preferred_element_type=jnp.float32)
    rE[...] = jnp.dot(r, we, preferred_element_type=jnp.float32)
    lH[...] = jnp.dot(q, w, preferred_element_type=jnp.float32)
    lE[...] = jnp.dot(q, we, preferred_element_type=jnp.float32)


def _node_body(fH, fE, rH, rE, lH, lE, i0_2d, i1_2d, ht_out, ed_out,
               fHb, fEb, rHb, rEb, lHb, lEb, i0b, i1b, htb, edb, sems):
    cid = lax.axis_index("c")
    sid = lax.axis_index("s")
    wid = sid * NC + cid

    def do_chunk(c):
        base = c * 64
        pltpu.sync_copy(i0_2d.at[c], i0b)
        pltpu.sync_copy(i1_2d.at[c], i1b)
        cps = [
            pltpu.async_copy(fH.at[pl.ds(base, 64)], fHb, sems.at[0]),
            pltpu.async_copy(fE.at[pl.ds(base, 64)], fEb, sems.at[1]),
            pltpu.async_copy(rH.at[i0b], rHb, sems.at[2]),
            pltpu.async_copy(rE.at[i0b], rEb, sems.at[3]),
            pltpu.async_copy(lH.at[i1b], lHb, sems.at[4]),
            pltpu.async_copy(lE.at[i1b], lEb, sems.at[5]),
        ]
        for cp in cps:
            cp.wait()

        @plsc.parallel_loop(0, 64, unroll=2)
        def _(n):
            for g in range(H):
                s = pl.ds(g * 16, 16)
                htb[n, s] = fHb[n, s] + rHb[n, s] + lHb[n, s]
            se = pl.ds(0, 16)
            htb[n, pl.ds(F, 16)] = fEb[n, se] + rEb[n, se] + lEb[n, se]
            sd = pl.ds(16, 16)
            edb[n, :] = fEb[n, sd] + rEb[n, sd] + lEb[n, sd]
        pltpu.sync_copy(htb, ht_out.at[pl.ds(base, 64)])
        pltpu.sync_copy(edb, ed_out.at[pl.ds(base, 64)])

    for t in range(NCH // 32):
        do_chunk(wid + t * 32)


def _edge_body(ht, ed, src2d, dst2d, part_out, hb0, hb1, hb2, db0, db1, db2,
               sb0, sb1, dk0, dk1, acc, gh, ge, ss, gi):
    cid = lax.axis_index("c")
    sid = lax.axis_index("s")
    wid = sid * NC + cid
    ntile = jnp.minimum(E // EC - wid * CPT, CPT)
    rowbase = wid * CPT
    hbs = (hb0, hb1, hb2)
    dbs = (db0, db1, db2)
    sbs = (sb0, sb1)
    dks = (dk0, dk1)

    zrow = jnp.zeros((NLANES,), jnp.float32)

    def zero_row(n, _):
        for g in range(DH // 16):
            hb0[n, pl.ds(g * 16, 16)] = zrow
        return 0

    lax.fori_loop(0, EC, zero_row, 0)
    for k in range(NPAD // NS // EC):
        pltpu.sync_copy(hb0, acc.at[pl.ds(sid * 640 + k * EC, EC)])
    plsc.subcore_barrier()

    def load_block(s, b):
        pltpu.async_copy(src2d.at[pl.ds(rowbase + 3 * s, 3)], sbs[b],
                         gi.at[2 * b])
        pltpu.async_copy(dst2d.at[pl.ds(rowbase + 3 * s, 3)], dks[b],
                         gi.at[2 * b + 1])

    def wait_block(b):
        pltpu.make_async_copy(src2d.at[pl.ds(0, 3)], sbs[b],
                              gi.at[2 * b]).wait()
        pltpu.make_async_copy(dst2d.at[pl.ds(0, 3)], dks[b],
                              gi.at[2 * b + 1]).wait()

    def start_gathers(t, k, b, r):
        @pl.when(t < ntile)
        def _():
            pltpu.async_copy(ht.at[sbs[b].at[r]], hbs[k], gh.at[k])
            pltpu.async_copy(ed.at[dks[b].at[r]], dbs[k], ge.at[k])

    def step(t, k, b, r):
        hb, db = hbs[k], dbs[k]

        @pl.when(t < ntile)
        def _():
            pltpu.make_async_copy(ht.at[dk0.at[0]], hb, gh.at[k]).wait()
            pltpu.make_async_copy(ed.at[dk0.at[0]], db, ge.at[k]).wait()

            @plsc.parallel_loop(0, EC, unroll=2)
            def _(e):
                vs = hb[e, pl.ds(F, 16)]
                vd = db[e, :]
                x = vs + vd
                x = jnp.maximum(x, x * 0.2)
                ex = jnp.exp(x)
                hb[e, pl.ds(F, 16)] = ex
                for g in range(H):
                    s = pl.ds(g * 16, 16)
                    hb[e, s] = hb[e, s] * ex
            pltpu.async_copy(hb, acc.at[dks[b].at[r]], ss.at[k], add=True)

        kp2 = (k + 2) % 3

        @pl.when((t >= 1) & (t - 1 < ntile))
        def _():
            pltpu.make_async_copy(hbs[kp2], acc.at[dk0.at[0]],
                                  ss.at[kp2]).wait()

    load_block(0, 0)
    wait_block(0)
    start_gathers(0, 0, 0, 0)
    start_gathers(1, 1, 0, 1)

    def pairbody(u, _):
        t0 = u * 6
        step(t0 + 0, 0, 0, 0)
        load_block(2 * u + 1, 1)
        start_gathers(t0 + 2, 2, 0, 2)
        step(t0 + 1, 1, 0, 1)
        wait_block(1)
        start_gathers(t0 + 3, 0, 1, 0)
        step(t0 + 2, 2, 0, 2)
        start_gathers(t0 + 4, 1, 1, 1)
        step(t0 + 3, 0, 1, 0)
        load_block(2 * u + 2, 0)
        start_gathers(t0 + 5, 2, 1, 2)
        step(t0 + 4, 1, 1, 1)
        wait_block(0)
        start_gathers(t0 + 6, 0, 0, 0)
        step(t0 + 5, 2, 1, 2)
        start_gathers(t0 + 7, 1, 0, 1)
        return 0

    lax.fori_loop(0, NSUP // 2, pairbody, 0)

    plsc.subcore_barrier()
    pltpu.sync_copy(acc.at[pl.ds(sid * 640, 640)],
                    part_out.at[cid, pl.ds(sid * 640, 640)])


def _finish_body(part, out):
    p = part[0] + part[1]
    num = p[:, :F]
    d16 = p[:, F:DH]
    i16 = lax.broadcasted_iota(jnp.int32, (L, F), 0)
    j = lax.broadcasted_iota(jnp.int32, (L, F), 1)
    r16 = ((i16 == (j % H)) & (i16 < H)).astype(jnp.float32)
    dd = jnp.dot(d16, r16, preferred_element_type=jnp.float32)
    a = num / (dd + 1e-16)
    jj = lax.broadcasted_iota(jnp.int32, (F, F), 0)
    mm = lax.broadcasted_iota(jnp.int32, (F, F), 1)
    perm = (jj == ((mm % FO) * H + mm // FO)).astype(jnp.float32)
    out[...] = jnp.maximum(jnp.dot(a, perm, preferred_element_type=jnp.float32),
                           0.0)


def kernel(lanes_feat, lane_ids, edge_index, road_emb, lane_emb, W, a_src,
           a_dst):
    f32 = jnp.float32
    lanes_feat = lanes_feat.astype(f32)
    wp = W.astype(f32).reshape(F, H, FO).transpose(0, 2, 1).reshape(F, F)
    asf = a_src.astype(f32).transpose(1, 0).reshape(1, F)
    adf = a_dst.astype(f32).transpose(1, 0).reshape(1, F)

    xp = jnp.zeros((NPAD, F), f32)
    xp = xp.at[:N, :L - 2].set(lanes_feat)
    rp = jnp.zeros((V, F), f32).at[:, L - 2:].set(road_emb.astype(f32))
    lpe = jnp.zeros((V, F), f32).at[:, L - 2:].set(lane_emb.astype(f32))

    ids = lane_ids.astype(jnp.int32)
    i0 = jnp.zeros((NPAD,), jnp.int32).at[:N].set(ids[:, 0]).reshape(NCH, 64)
    i1 = jnp.zeros((NPAD,), jnp.int32).at[:N].set(ids[:, 1]).reshape(NCH, 64)
    epad = ERPAD * EC - E
    src2d = jnp.pad(edge_index[0].astype(jnp.int32), (0, epad)).reshape(
        ERPAD, EC)
    dst2d = jnp.pad(edge_index[1].astype(jnp.int32), (0, epad)).reshape(
        ERPAD, EC)

    fH, fE, rH, rE, lH, lE = pl.pallas_call(
        _dense_body,
        out_shape=[
            jax.ShapeDtypeStruct((NPAD, F), f32),
            jax.ShapeDtypeStruct((NPAD, F), f32),
            jax.ShapeDtypeStruct((V, F), f32),
            jax.ShapeDtypeStruct((V, F), f32),
            jax.ShapeDtypeStruct((V, F), f32),
            jax.ShapeDtypeStruct((V, F), f32),
        ],
    )(xp, rp, lpe, wp, asf, adf)

    mesh = plsc.VectorSubcoreMesh(core_axis_name="c", subcore_axis_name="s")

    sc_params = pltpu.CompilerParams(use_tc_tiling_on_sc=False)
    node_k = pl.kernel(
        _node_body,
        out_type=[
            jax.ShapeDtypeStruct((NPAD, DH), f32),
            jax.ShapeDtypeStruct((NPAD, L), f32),
        ],
        mesh=mesh,
        compiler_params=sc_params,
        scratch_types=[
            pltpu.VMEM((64, F), f32),
            pltpu.VMEM((64, F), f32),
            pltpu.VMEM((64, F), f32),
            pltpu.VMEM((64, F), f32),
            pltpu.VMEM((64, F), f32),
            pltpu.VMEM((64, F), f32),
            pltpu.VMEM((64,), jnp.int32),
            pltpu.VMEM((64,), jnp.int32),
            pltpu.VMEM((64, DH), f32),
            pltpu.VMEM((64, L), f32),
            pltpu.SemaphoreType.DMA((6,)),
        ],
    )
    ht, edt = node_k(fH, fE, rH, rE, lH, lE, i0, i1)

    edge_k = pl.kernel(
        _edge_body,
        out_type=[jax.ShapeDtypeStruct((NC, NPAD, DH), f32)],
        mesh=mesh,
        compiler_params=sc_params,
        scratch_types=[
            pltpu.VMEM((EC, DH), f32),
            pltpu.VMEM((EC, DH), f32),
            pltpu.VMEM((EC, DH), f32),
            pltpu.VMEM((EC, L), f32),
            pltpu.VMEM((EC, L), f32),
            pltpu.VMEM((EC, L), f32),
            pltpu.VMEM((3, EC), jnp.int32),
            pltpu.VMEM((3, EC), jnp.int32),
            pltpu.VMEM((3, EC), jnp.int32),
            pltpu.VMEM((3, EC), jnp.int32),
            pltpu.VMEM_SHARED((NPAD, DH), f32),
            pltpu.SemaphoreType.DMA((3,)),
            pltpu.SemaphoreType.DMA((3,)),
            pltpu.SemaphoreType.DMA((3,)),
            pltpu.SemaphoreType.DMA((4,)),
        ],
    )
    (part,) = edge_k(ht, edt, src2d, dst2d)

    out = pl.pallas_call(
        _finish_body,
        grid=(NPAD // 512,),
        in_specs=[pl.BlockSpec((NC, 512, DH), lambda i: (0, i, 0))],
        out_specs=pl.BlockSpec((512, F), lambda i: (i, 0)),
        out_shape=jax.ShapeDtypeStruct((NPAD, F), f32),
    )(part)
    return out[:N]
